# baseline (speedup 1.0000x reference)
import functools

import jax
import jax.numpy as jnp
import numpy as np
from jax import lax
from jax.experimental import pallas as pl
from jax.experimental.pallas import tpu as pltpu
from jax.experimental.pallas import tpu_sc as plsc

_NB = 8
_SP = 24
_N = 2048
_TM = 1152
_NW = 32


def _knn_select_kernel(pc_ref, gct_ref, idx_ref, w_ref):
    pc = pc_ref[0]
    gct = gct_ref[0]
    pn = jnp.sum(pc * pc, axis=1, keepdims=True)
    gn = jnp.sum(gct * gct, axis=0, keepdims=True)
    dot = jax.lax.dot_general(
        pc, gct, (((1,), (0,)), ((), ())),
        precision=jax.lax.Precision.DEFAULT,
        preferred_element_type=jnp.float32)
    work = (pn + gn) - 2.0 * dot
    col = jax.lax.broadcasted_iota(jnp.int32, work.shape, 0)
    packed = jax.lax.bitcast_convert_type(
        (jax.lax.bitcast_convert_type(work, jnp.int32) & ~0x7FF) | col,
        jnp.float32)
    sel_tb = work[:_N // 2] <= work[_N // 2:]
    lo = jnp.where(sel_tb, work[:_N // 2], work[_N // 2:])
    hi = jnp.where(sel_tb, work[_N // 2:], work[:_N // 2])
    lop = jnp.where(sel_tb, packed[:_N // 2], packed[_N // 2:])
    hip = jnp.where(sel_tb, packed[_N // 2:], packed[:_N // 2])
    inf = jnp.float32(jnp.inf)
    for i in range(_NB):
        m = jnp.min(lo, axis=0, keepdims=True)
        sel = lo == m
        pm = jnp.min(jnp.where(sel, lop, inf), axis=0, keepdims=True)
        mb = jax.lax.bitcast_convert_type(pm, jnp.int32)
        wi = 1.0 / jnp.square(jnp.sqrt(jnp.maximum(m, 0.0)) + 1e-8)
        idx_ref[0, i:i + 1, :] = mb & 0x7FF
        w_ref[0, i:i + 1, :] = wi
        lo = jnp.where(sel, hi, lo)
        hi = jnp.where(sel, inf, hi)
        lop = jnp.where(sel, hip, lop)
        hip = jnp.where(sel, inf, hip)


def _sc_combine(idx_hbm, w_hbm, flow_hbm, out_hbm, idx_v, w_v, flow_v, out_v):
    nq = idx_hbm.shape[0] // _NB
    m = _SP ** 3
    chunk = nq // _NW
    wid = lax.axis_index("s") * 2 + lax.axis_index("c")
    base = wid * chunk
    b = base // m
    off = base - b * m
    pltpu.sync_copy(flow_hbm.at[pl.ds(b * 3 * _N, 3 * _N)], flow_v)
    for k in range(_NB):
        src = (b * _NB + k) * m + off
        pltpu.sync_copy(idx_hbm.at[pl.ds(src, chunk)],
                        idx_v.at[pl.ds(k * chunk, chunk)])
        pltpu.sync_copy(w_hbm.at[pl.ds(src, chunk)],
                        w_v.at[pl.ds(k * chunk, chunk)])

    def body(j, carry):
        q = j * 16
        acc_x = jnp.zeros((16,), jnp.float32)
        acc_y = jnp.zeros((16,), jnp.float32)
        acc_z = jnp.zeros((16,), jnp.float32)
        dsum = jnp.zeros((16,), jnp.float32)
        for k in range(_NB):
            ik = idx_v[pl.ds(k * chunk + q, 16)]
            wk = w_v[pl.ds(k * chunk + q, 16)]
            gx = plsc.load_gather(flow_v, [ik])
            gy = plsc.load_gather(flow_v, [ik + _N])
            gz = plsc.load_gather(flow_v, [ik + 2 * _N])
            acc_x = acc_x + wk * gx
            acc_y = acc_y + wk * gy
            acc_z = acc_z + wk * gz
            dsum = dsum + wk
        inv = 1.0 / dsum
        out_v[pl.ds(q, 16)] = acc_x * inv
        out_v[pl.ds(chunk + q, 16)] = acc_y * inv
        out_v[pl.ds(2 * chunk + q, 16)] = acc_z * inv
        return carry

    lax.fori_loop(0, chunk // 16, body, 0)
    for c in range(3):
        pltpu.sync_copy(out_v.at[pl.ds(c * chunk, chunk)],
                        out_hbm.at[pl.ds(c * nq + base, chunk)])


def _div_kernel(fx_ref, fy_ref, fz_ref, out_ref):
    h = 2.0 * np.pi / _SP
    fx = fx_ref[...]
    fy = fy_ref[...]
    fz = fz_ref[...]
    row = jax.lax.broadcasted_iota(jnp.int32, fx.shape, 0)
    y = row % _SP
    x = (row // _SP) % _SP
    upx = jnp.concatenate([fx[_SP:], fx[-_SP:]], axis=0)
    dnx = jnp.concatenate([fx[:_SP], fx[:-_SP]], axis=0)
    gx = (upx - dnx) / (2.0 * h)
    gx = jnp.where(x == 0, (upx - fx) / h, gx)
    gx = jnp.where(x == _SP - 1, (fx - dnx) / h, gx)
    upy = jnp.concatenate([fy[1:], fy[-1:]], axis=0)
    dny = jnp.concatenate([fy[:1], fy[:-1]], axis=0)
    gy = (upy - dny) / (2.0 * h)
    gy = jnp.where(y == 0, (upy - fy) / h, gy)
    gy = jnp.where(y == _SP - 1, (fy - dny) / h, gy)
    zc = (fz[:, 2:] - fz[:, :-2]) / (2.0 * h)
    z0 = (fz[:, 1:2] - fz[:, 0:1]) / h
    z1 = (fz[:, -1:] - fz[:, -2:-1]) / h
    gz = jnp.concatenate([z0, zc, z1], axis=1)
    div = gx + gy + gz
    out_ref[...] = jnp.broadcast_to(jnp.mean(jnp.abs(div)), (1, 1))


def kernel(flow, coords, grid_coords):
    B, N, _ = coords.shape
    M = grid_coords.shape[1]
    gct_all = jnp.transpose(grid_coords, (0, 2, 1))
    nt = M // _TM
    idx, w = pl.pallas_call(
        _knn_select_kernel,
        grid=(B, nt),
        in_specs=[
            pl.BlockSpec((1, N, 3), lambda b, i: (b, 0, 0)),
            pl.BlockSpec((1, 3, _TM), lambda b, i: (b, 0, i)),
        ],
        out_specs=[
            pl.BlockSpec((1, _NB, _TM), lambda b, i: (b, 0, i)),
            pl.BlockSpec((1, _NB, _TM), lambda b, i: (b, 0, i)),
        ],
        out_shape=[
            jax.ShapeDtypeStruct((B, _NB, M), jnp.int32),
            jax.ShapeDtypeStruct((B, _NB, M), jnp.float32),
        ],
    )(coords, gct_all)
    idx_t = idx.reshape(-1)
    w_t = w.reshape(-1)
    flow_flat = jnp.transpose(flow, (0, 2, 1)).reshape(-1)

    mesh = plsc.VectorSubcoreMesh(core_axis_name="c", subcore_axis_name="s")
    chunk = (B * M) // _NW
    sc = functools.partial(
        pl.kernel, mesh=mesh,
        compiler_params=pltpu.CompilerParams(needs_layout_passes=False),
        out_type=jax.ShapeDtypeStruct((3 * B * M,), jnp.float32),
        scratch_types=[
            pltpu.VMEM((_NB * chunk,), jnp.int32),
            pltpu.VMEM((_NB * chunk,), jnp.float32),
            pltpu.VMEM((3 * N,), jnp.float32),
            pltpu.VMEM((3 * chunk,), jnp.float32),
        ],
    )(_sc_combine)
    gf_t = sc(idx_t, w_t, flow_flat).reshape(3, B * M)

    fx = gf_t[0].reshape(B * _SP * _SP, _SP)
    fy = gf_t[1].reshape(B * _SP * _SP, _SP)
    fz = gf_t[2].reshape(B * _SP * _SP, _SP)
    out = pl.pallas_call(
        _div_kernel,
        out_shape=jax.ShapeDtypeStruct((1, 1), jnp.float32),
    )(fx, fy, fz)
    return out[0, 0]


# speedup vs baseline: 1.8646x; 1.8646x over previous
import functools

import jax
import jax.numpy as jnp
import numpy as np
from jax import lax
from jax.experimental import pallas as pl
from jax.experimental.pallas import tpu as pltpu
from jax.experimental.pallas import tpu_sc as plsc

_NB = 8
_SP = 24
_N = 2048
_TM = 1152
_NW = 32


def _knn_select_kernel(pc_ref, gct_ref, idx_ref, w_ref):
    pc = pc_ref[0]
    gct = gct_ref[0]
    pn = jnp.sum(pc * pc, axis=1, keepdims=True)
    gn = jnp.sum(gct * gct, axis=0, keepdims=True)
    dot = jax.lax.dot_general(
        pc, gct, (((1,), (0,)), ((), ())),
        precision=jax.lax.Precision.DEFAULT,
        preferred_element_type=jnp.float32)
    work = (pn + gn) - 2.0 * dot
    col = jax.lax.broadcasted_iota(jnp.int32, work.shape, 0)
    wb = jax.lax.bitcast_convert_type(work, jnp.int32)
    work = jax.lax.bitcast_convert_type((wb & ~0x7FF) | col, jnp.float32)
    top = work[:_N // 2]
    bot = work[_N // 2:]
    lo = jnp.minimum(top, bot)
    hi = jnp.maximum(top, bot)
    for i in range(_NB):
        m = jnp.min(lo, axis=0, keepdims=True)
        sel = lo == m
        mb = jax.lax.bitcast_convert_type(m, jnp.int32)
        mval = jax.lax.bitcast_convert_type(mb & ~0x7FF, jnp.float32)
        wi = 1.0 / jnp.square(jnp.sqrt(jnp.maximum(mval, 0.0)) + 1e-8)
        idx_ref[0, i:i + 1, :] = mb & 0x7FF
        w_ref[0, i:i + 1, :] = wi
        lo = jnp.where(sel, hi, lo)
        hi = jnp.where(sel, jnp.float32(jnp.inf), hi)


def _sc_combine(idx_hbm, w_hbm, flow_hbm, out_hbm, idx_v, w_v, flow_v, out_v):
    nq = idx_hbm.shape[0] // _NB
    m = _SP ** 3
    chunk = nq // _NW
    wid = lax.axis_index("s") * 2 + lax.axis_index("c")
    base = wid * chunk
    b = base // m
    off = base - b * m
    pltpu.sync_copy(flow_hbm.at[pl.ds(b * 3 * _N, 3 * _N)], flow_v)
    for k in range(_NB):
        src = (b * _NB + k) * m + off
        pltpu.sync_copy(idx_hbm.at[pl.ds(src, chunk)],
                        idx_v.at[pl.ds(k * chunk, chunk)])
        pltpu.sync_copy(w_hbm.at[pl.ds(src, chunk)],
                        w_v.at[pl.ds(k * chunk, chunk)])

    def body(j, carry):
        q = j * 16
        acc_x = jnp.zeros((16,), jnp.float32)
        acc_y = jnp.zeros((16,), jnp.float32)
        acc_z = jnp.zeros((16,), jnp.float32)
        dsum = jnp.zeros((16,), jnp.float32)
        for k in range(_NB):
            ik = idx_v[pl.ds(k * chunk + q, 16)]
            wk = w_v[pl.ds(k * chunk + q, 16)]
            gx = plsc.load_gather(flow_v, [ik])
            gy = plsc.load_gather(flow_v, [ik + _N])
            gz = plsc.load_gather(flow_v, [ik + 2 * _N])
            acc_x = acc_x + wk * gx
            acc_y = acc_y + wk * gy
            acc_z = acc_z + wk * gz
            dsum = dsum + wk
        inv = 1.0 / dsum
        out_v[pl.ds(q, 16)] = acc_x * inv
        out_v[pl.ds(chunk + q, 16)] = acc_y * inv
        out_v[pl.ds(2 * chunk + q, 16)] = acc_z * inv
        return carry

    lax.fori_loop(0, chunk // 16, body, 0)
    for c in range(3):
        pltpu.sync_copy(out_v.at[pl.ds(c * chunk, chunk)],
                        out_hbm.at[pl.ds(c * nq + base, chunk)])


def _div_kernel(fx_ref, fy_ref, fz_ref, out_ref):
    h = 2.0 * np.pi / _SP
    fx = fx_ref[...]
    fy = fy_ref[...]
    fz = fz_ref[...]
    row = jax.lax.broadcasted_iota(jnp.int32, fx.shape, 0)
    y = row % _SP
    x = (row // _SP) % _SP
    upx = jnp.concatenate([fx[_SP:], fx[-_SP:]], axis=0)
    dnx = jnp.concatenate([fx[:_SP], fx[:-_SP]], axis=0)
    gx = (upx - dnx) / (2.0 * h)
    gx = jnp.where(x == 0, (upx - fx) / h, gx)
    gx = jnp.where(x == _SP - 1, (fx - dnx) / h, gx)
    upy = jnp.concatenate([fy[1:], fy[-1:]], axis=0)
    dny = jnp.concatenate([fy[:1], fy[:-1]], axis=0)
    gy = (upy - dny) / (2.0 * h)
    gy = jnp.where(y == 0, (upy - fy) / h, gy)
    gy = jnp.where(y == _SP - 1, (fy - dny) / h, gy)
    zc = (fz[:, 2:] - fz[:, :-2]) / (2.0 * h)
    z0 = (fz[:, 1:2] - fz[:, 0:1]) / h
    z1 = (fz[:, -1:] - fz[:, -2:-1]) / h
    gz = jnp.concatenate([z0, zc, z1], axis=1)
    div = gx + gy + gz
    out_ref[...] = jnp.broadcast_to(jnp.mean(jnp.abs(div)), (1, 1))


def kernel(flow, coords, grid_coords):
    B, N, _ = coords.shape
    M = grid_coords.shape[1]
    gct_all = jnp.transpose(grid_coords, (0, 2, 1))
    nt = M // _TM
    idx, w = pl.pallas_call(
        _knn_select_kernel,
        grid=(B, nt),
        in_specs=[
            pl.BlockSpec((1, N, 3), lambda b, i: (b, 0, 0)),
            pl.BlockSpec((1, 3, _TM), lambda b, i: (b, 0, i)),
        ],
        out_specs=[
            pl.BlockSpec((1, _NB, _TM), lambda b, i: (b, 0, i)),
            pl.BlockSpec((1, _NB, _TM), lambda b, i: (b, 0, i)),
        ],
        out_shape=[
            jax.ShapeDtypeStruct((B, _NB, M), jnp.int32),
            jax.ShapeDtypeStruct((B, _NB, M), jnp.float32),
        ],
    )(coords, gct_all)
    idx_t = idx.reshape(-1)
    w_t = w.reshape(-1)
    flow_flat = jnp.transpose(flow, (0, 2, 1)).reshape(-1)

    mesh = plsc.VectorSubcoreMesh(core_axis_name="c", subcore_axis_name="s")
    chunk = (B * M) // _NW
    sc = functools.partial(
        pl.kernel, mesh=mesh,
        compiler_params=pltpu.CompilerParams(needs_layout_passes=False),
        out_type=jax.ShapeDtypeStruct((3 * B * M,), jnp.float32),
        scratch_types=[
            pltpu.VMEM((_NB * chunk,), jnp.int32),
            pltpu.VMEM((_NB * chunk,), jnp.float32),
            pltpu.VMEM((3 * N,), jnp.float32),
            pltpu.VMEM((3 * chunk,), jnp.float32),
        ],
    )(_sc_combine)
    gf_t = sc(idx_t, w_t, flow_flat).reshape(3, B * M)

    fx = gf_t[0].reshape(B * _SP * _SP, _SP)
    fy = gf_t[1].reshape(B * _SP * _SP, _SP)
    fz = gf_t[2].reshape(B * _SP * _SP, _SP)
    out = pl.pallas_call(
        _div_kernel,
        out_shape=jax.ShapeDtypeStruct((1, 1), jnp.float32),
    )(fx, fy, fz)
    return out[0, 0]
